# Initial kernel scaffold; baseline (speedup 1.0000x reference)
import jax
import jax.numpy as jnp
from jax.experimental import pallas as pl


def kernel(x, edge_index, W1, b1, W2, b2):
    raise NotImplementedError("write your pallas kernel here")


# speedup vs baseline: 43.7669x; 43.7669x over previous
import functools

import jax
import jax.numpy as jnp
from jax import lax
from jax.experimental import pallas as pl
from jax.experimental.pallas import tpu as pltpu
from jax.experimental.pallas import tpu_sc as plsc

N = 50000
E = 1600000
D_IN = 1433
D_HID = 16

NUM_CORES = 2
NUM_SUBCORES = 16
NUM_WORKERS = NUM_CORES * NUM_SUBCORES
EDGES_PER_WORKER = E // NUM_WORKERS
CHUNK = 2000
NUM_CHUNKS = EDGES_PER_WORKER // CHUNK
ROWS_PER_TILE = 3128
NP = ROWS_PER_TILE * NUM_SUBCORES
ROWS_TAIL = ROWS_PER_TILE - CHUNK


def _sc_mesh():
    return plsc.VectorSubcoreMesh(core_axis_name="c", subcore_axis_name="s")


_SC_PARAMS = pltpu.CompilerParams(use_tc_tiling_on_sc=False)


def _make_deg_kernel():
    @functools.partial(
        pl.kernel,
        out_type=jax.ShapeDtypeStruct((NUM_CORES * NP,), jnp.float32),
        mesh=_sc_mesh(),
        compiler_params=_SC_PARAMS,
        scratch_types=[
            pltpu.VMEM((CHUNK,), jnp.int32),
            pltpu.VMEM((CHUNK,), jnp.float32),
            pltpu.VMEM((ROWS_PER_TILE,), jnp.float32),
            pltpu.VMEM_SHARED((NP,), jnp.float32),
        ],
    )
    def deg_kernel(dst_hbm, zeros_hbm, ones_hbm, out_hbm, dst_v, ones_v, tmp_v, acc_sh):
        cid = lax.axis_index("c")
        sid = lax.axis_index("s")
        wid = cid * NUM_SUBCORES + sid
        row0 = sid * ROWS_PER_TILE
        pltpu.sync_copy(zeros_hbm.at[pl.ds(row0, ROWS_PER_TILE)], tmp_v)
        pltpu.sync_copy(tmp_v, acc_sh.at[pl.ds(row0, ROWS_PER_TILE)])
        pltpu.sync_copy(ones_hbm, ones_v)
        plsc.subcore_barrier()

        def body(j, carry):
            base = wid * EDGES_PER_WORKER + j * CHUNK
            pltpu.sync_copy(dst_hbm.at[pl.ds(base, CHUNK)], dst_v)
            pltpu.sync_copy(ones_v, acc_sh.at[dst_v], add=True)
            return carry

        lax.fori_loop(0, NUM_CHUNKS, body, 0)
        plsc.subcore_barrier()
        pltpu.sync_copy(acc_sh.at[pl.ds(row0, ROWS_PER_TILE)], tmp_v)
        pltpu.sync_copy(tmp_v, out_hbm.at[pl.ds(cid * NP + row0, ROWS_PER_TILE)])

    return deg_kernel


def _make_scatter_kernel(d):
    @functools.partial(
        pl.kernel,
        out_type=jax.ShapeDtypeStruct((NUM_CORES, NP, d), jnp.float32),
        mesh=_sc_mesh(),
        compiler_params=_SC_PARAMS,
        scratch_types=[
            pltpu.VMEM((CHUNK,), jnp.int32),
            pltpu.VMEM((CHUNK,), jnp.int32),
            pltpu.VMEM((CHUNK, d), jnp.float32),
            pltpu.VMEM_SHARED((NP, d), jnp.float32),
            pltpu.SemaphoreType.DMA,
        ],
    )
    def scatter_kernel(
        hs_hbm, src_hbm, dst_hbm, zeros_hbm, out_hbm,
        src_v, dst_v, rows_v, acc_sh, sem
    ):
        cid = lax.axis_index("c")
        sid = lax.axis_index("s")
        wid = cid * NUM_SUBCORES + sid
        row0 = sid * ROWS_PER_TILE
        pltpu.sync_copy(zeros_hbm, rows_v)
        pltpu.sync_copy(rows_v, acc_sh.at[pl.ds(row0, CHUNK)])
        pltpu.sync_copy(
            rows_v.at[pl.ds(0, ROWS_TAIL)], acc_sh.at[pl.ds(row0 + CHUNK, ROWS_TAIL)]
        )
        plsc.subcore_barrier()

        def body(j, carry):
            base = wid * EDGES_PER_WORKER + j * CHUNK
            pltpu.sync_copy(src_hbm.at[pl.ds(base, CHUNK)], src_v)
            pltpu.sync_copy(dst_hbm.at[pl.ds(base, CHUNK)], dst_v)
            pltpu.async_copy(hs_hbm.at[src_v], rows_v, sem).wait()
            pltpu.sync_copy(rows_v, acc_sh.at[dst_v], add=True)
            return carry

        lax.fori_loop(0, NUM_CHUNKS, body, 0)
        plsc.subcore_barrier()
        pltpu.sync_copy(acc_sh.at[pl.ds(row0, CHUNK)], rows_v)
        pltpu.sync_copy(rows_v, out_hbm.at[cid, pl.ds(row0, CHUNK)])
        pltpu.sync_copy(
            acc_sh.at[pl.ds(row0 + CHUNK, ROWS_TAIL)], rows_v.at[pl.ds(0, ROWS_TAIL)]
        )
        pltpu.sync_copy(
            rows_v.at[pl.ds(0, ROWS_TAIL)], out_hbm.at[cid, pl.ds(row0 + CHUNK, ROWS_TAIL)]
        )

    return scatter_kernel


_BN = 2000
_GRID = N // _BN


def _mm_body(x_ref, w_ref, o_ref):
    o_ref[...] = jnp.dot(
        x_ref[...], w_ref[...], preferred_element_type=jnp.float32
    )


def _tc_matmul(x, w1):
    bn = 1000
    return pl.pallas_call(
        _mm_body,
        grid=(N // bn,),
        in_specs=[
            pl.BlockSpec((bn, D_IN), lambda i: (i, 0)),
            pl.BlockSpec((D_IN, D_HID), lambda i: (0, 0)),
        ],
        out_specs=pl.BlockSpec((bn, D_HID), lambda i: (i, 0)),
        out_shape=jax.ShapeDtypeStruct((N, D_HID), jnp.float32),
    )(x, w1)


def _scale_body(h_ref, d0_ref, d1_ref, o_ref):
    deg = d0_ref[...] + d1_ref[...] + 1.0
    o_ref[...] = h_ref[...] * lax.rsqrt(deg)


def _tc_scale(h1, d0, d1):
    return pl.pallas_call(
        _scale_body,
        grid=(_GRID,),
        in_specs=[
            pl.BlockSpec((_BN, D_HID), lambda i: (i, 0)),
            pl.BlockSpec((_BN, 1), lambda i: (i, 0)),
            pl.BlockSpec((_BN, 1), lambda i: (i, 0)),
        ],
        out_specs=pl.BlockSpec((_BN, D_HID), lambda i: (i, 0)),
        out_shape=jax.ShapeDtypeStruct((N, D_HID), jnp.float32),
    )(h1, d0, d1)


def _mid_body(h1_ref, p0_ref, p1_ref, d0_ref, d1_ref, b1_ref, w2_ref, h2_ref, hs2_ref):
    deg = d0_ref[...] + d1_ref[...] + 1.0
    invd = 1.0 / deg
    dis = lax.rsqrt(deg)
    pre = (p0_ref[...] + p1_ref[...]) * dis + h1_ref[...] * invd + b1_ref[...]
    a = jnp.maximum(pre, 0.0)
    h2 = jnp.dot(a, w2_ref[...], preferred_element_type=jnp.float32)
    h2_ref[...] = h2
    hs2_ref[...] = h2 * dis


def _tc_mid(h1, p0, p1, d0, d1, b1_2d, w2p):
    return pl.pallas_call(
        _mid_body,
        grid=(_GRID,),
        in_specs=[
            pl.BlockSpec((_BN, D_HID), lambda i: (i, 0)),
            pl.BlockSpec((_BN, D_HID), lambda i: (i, 0)),
            pl.BlockSpec((_BN, D_HID), lambda i: (i, 0)),
            pl.BlockSpec((_BN, 1), lambda i: (i, 0)),
            pl.BlockSpec((_BN, 1), lambda i: (i, 0)),
            pl.BlockSpec((1, D_HID), lambda i: (0, 0)),
            pl.BlockSpec((D_HID, 8), lambda i: (0, 0)),
        ],
        out_specs=[
            pl.BlockSpec((_BN, 8), lambda i: (i, 0)),
            pl.BlockSpec((_BN, 8), lambda i: (i, 0)),
        ],
        out_shape=[
            jax.ShapeDtypeStruct((N, 8), jnp.float32),
            jax.ShapeDtypeStruct((N, 8), jnp.float32),
        ],
    )(h1, p0, p1, d0, d1, b1_2d, w2p)


def _final_body(h2_ref, q0_ref, q1_ref, d0_ref, d1_ref, b2_ref, o_ref):
    deg = d0_ref[...] + d1_ref[...] + 1.0
    invd = 1.0 / deg
    dis = lax.rsqrt(deg)
    l = (q0_ref[...] + q1_ref[...]) * dis + h2_ref[...] * invd + b2_ref[...]
    col = lax.broadcasted_iota(jnp.int32, l.shape, 1)
    valid = col < 7
    lm = jnp.where(valid, l, -1e30)
    m = jnp.max(lm, axis=1, keepdims=True)
    e = jnp.where(valid, jnp.exp(l - m), 0.0)
    s = jnp.sum(e, axis=1, keepdims=True)
    out = l - m - jnp.log(s)
    o_ref[...] = out[:, :7]


def _tc_final(h2, q0, q1, d0, d1, b2_2d):
    return pl.pallas_call(
        _final_body,
        grid=(_GRID,),
        in_specs=[
            pl.BlockSpec((_BN, 8), lambda i: (i, 0)),
            pl.BlockSpec((_BN, 8), lambda i: (i, 0)),
            pl.BlockSpec((_BN, 8), lambda i: (i, 0)),
            pl.BlockSpec((_BN, 1), lambda i: (i, 0)),
            pl.BlockSpec((_BN, 1), lambda i: (i, 0)),
            pl.BlockSpec((1, 8), lambda i: (0, 0)),
        ],
        out_specs=pl.BlockSpec((_BN, 7), lambda i: (i, 0)),
        out_shape=jax.ShapeDtypeStruct((N, 7), jnp.float32),
    )(h2, q0, q1, d0, d1, b2_2d)


def kernel(x, edge_index, W1, b1, W2, b2):
    edge_index = edge_index.astype(jnp.int32)
    src = edge_index[0]
    dst = edge_index[1]

    zeros1 = jnp.zeros((NP,), jnp.float32)
    zeros16 = jnp.zeros((CHUNK, D_HID), jnp.float32)
    zeros8 = jnp.zeros((CHUNK, 8), jnp.float32)
    ones_c = jnp.ones((CHUNK,), jnp.float32)

    degs = _make_deg_kernel()(dst, zeros1, ones_c).reshape(NUM_CORES, NP)
    d0 = degs[0, :N][:, None]
    d1 = degs[1, :N][:, None]

    h1 = _tc_matmul(x, W1)
    hs1 = _tc_scale(h1, d0, d1)

    p = _make_scatter_kernel(D_HID)(hs1, src, dst, zeros16)
    p0 = p[0, :N]
    p1 = p[1, :N]

    w2p = jnp.concatenate([W2, jnp.zeros((D_HID, 1), jnp.float32)], axis=1)
    b1_2d = b1[None, :]
    h2, hs2 = _tc_mid(h1, p0, p1, d0, d1, b1_2d, w2p)

    q = _make_scatter_kernel(8)(hs2, src, dst, zeros8)
    q0 = q[0, :N]
    q1 = q[1, :N]

    b2_2d = jnp.concatenate([b2, jnp.zeros((1,), jnp.float32)])[None, :]
    return _tc_final(h2, q0, q1, d0, d1, b2_2d)


# speedup vs baseline: 49.8331x; 1.1386x over previous
import functools

import jax
import jax.numpy as jnp
from jax import lax
from jax.experimental import pallas as pl
from jax.experimental.pallas import tpu as pltpu
from jax.experimental.pallas import tpu_sc as plsc

N = 50000
E = 1600000
D_IN = 1433
D_HID = 16

NUM_CORES = 2
NUM_SUBCORES = 16
NUM_WORKERS = NUM_CORES * NUM_SUBCORES
EDGES_PER_WORKER = E // NUM_WORKERS
CHUNK = 2000
NUM_CHUNKS = EDGES_PER_WORKER // CHUNK
ROWS_PER_TILE = 3128
NP = ROWS_PER_TILE * NUM_SUBCORES
ROWS_TAIL = ROWS_PER_TILE - CHUNK


def _sc_mesh():
    return plsc.VectorSubcoreMesh(core_axis_name="c", subcore_axis_name="s")


_SC_PARAMS = pltpu.CompilerParams(use_tc_tiling_on_sc=False)


def _make_deg_kernel():
    @functools.partial(
        pl.kernel,
        out_type=jax.ShapeDtypeStruct((NUM_CORES * NP,), jnp.float32),
        mesh=_sc_mesh(),
        compiler_params=_SC_PARAMS,
        scratch_types=[
            pltpu.VMEM((CHUNK,), jnp.int32),
            pltpu.VMEM((CHUNK,), jnp.int32),
            pltpu.VMEM((CHUNK,), jnp.float32),
            pltpu.VMEM((ROWS_PER_TILE,), jnp.float32),
            pltpu.VMEM_SHARED((NP,), jnp.float32),
            pltpu.SemaphoreType.DMA,
            pltpu.SemaphoreType.DMA,
        ],
    )
    def deg_kernel(
        dst_hbm, zeros_hbm, ones_hbm, out_hbm,
        dst_v0, dst_v1, ones_v, tmp_v, acc_sh, s0, s1
    ):
        cid = lax.axis_index("c")
        sid = lax.axis_index("s")
        wid = cid * NUM_SUBCORES + sid
        row0 = sid * ROWS_PER_TILE
        pltpu.sync_copy(zeros_hbm.at[pl.ds(row0, ROWS_PER_TILE)], tmp_v)
        pltpu.sync_copy(tmp_v, acc_sh.at[pl.ds(row0, ROWS_PER_TILE)])
        pltpu.sync_copy(ones_hbm, ones_v)
        plsc.subcore_barrier()

        dsts = (dst_v0, dst_v1)
        sems = (s0, s1)
        s_desc = [None, None]

        def stage(j, b):
            base = wid * EDGES_PER_WORKER + j * CHUNK
            pltpu.sync_copy(dst_hbm.at[pl.ds(base, CHUNK)], dsts[b])

        stage(0, 0)
        for j in range(NUM_CHUNKS):
            b = j & 1
            nb = 1 - b
            if j + 1 < NUM_CHUNKS:
                if s_desc[nb] is not None:
                    s_desc[nb].wait()
                stage(j + 1, nb)
            s_desc[b] = pltpu.async_copy(
                ones_v, acc_sh.at[dsts[b]], sems[b], add=True
            )
        for d in s_desc:
            if d is not None:
                d.wait()
        plsc.subcore_barrier()
        pltpu.sync_copy(acc_sh.at[pl.ds(row0, ROWS_PER_TILE)], tmp_v)
        pltpu.sync_copy(tmp_v, out_hbm.at[pl.ds(cid * NP + row0, ROWS_PER_TILE)])

    return deg_kernel


def _make_scatter_kernel(d):
    @functools.partial(
        pl.kernel,
        out_type=jax.ShapeDtypeStruct((NUM_CORES, NP, d), jnp.float32),
        mesh=_sc_mesh(),
        compiler_params=_SC_PARAMS,
        scratch_types=[
            pltpu.VMEM((CHUNK,), jnp.int32),
            pltpu.VMEM((CHUNK,), jnp.int32),
            pltpu.VMEM((CHUNK, d), jnp.float32),
            pltpu.VMEM((CHUNK,), jnp.int32),
            pltpu.VMEM((CHUNK,), jnp.int32),
            pltpu.VMEM((CHUNK, d), jnp.float32),
            pltpu.VMEM_SHARED((NP, d), jnp.float32),
            pltpu.SemaphoreType.DMA,
            pltpu.SemaphoreType.DMA,
            pltpu.SemaphoreType.DMA,
            pltpu.SemaphoreType.DMA,
        ],
    )
    def scatter_kernel(
        hs_hbm, src_hbm, dst_hbm, zeros_hbm, out_hbm,
        src_v0, dst_v0, rows_v0, src_v1, dst_v1, rows_v1, acc_sh, g0, g1, s0, s1
    ):
        cid = lax.axis_index("c")
        sid = lax.axis_index("s")
        wid = cid * NUM_SUBCORES + sid
        row0 = sid * ROWS_PER_TILE
        pltpu.sync_copy(zeros_hbm, rows_v0)
        pltpu.sync_copy(rows_v0, acc_sh.at[pl.ds(row0, CHUNK)])
        pltpu.sync_copy(
            rows_v0.at[pl.ds(0, ROWS_TAIL)], acc_sh.at[pl.ds(row0 + CHUNK, ROWS_TAIL)]
        )
        plsc.subcore_barrier()

        srcs = (src_v0, src_v1)
        dsts = (dst_v0, dst_v1)
        rows = (rows_v0, rows_v1)
        gsems = (g0, g1)
        ssems = (s0, s1)
        g_desc = [None, None]
        s_desc = [None, None]

        def stage(j, b):
            base = wid * EDGES_PER_WORKER + j * CHUNK
            pltpu.sync_copy(src_hbm.at[pl.ds(base, CHUNK)], srcs[b])
            pltpu.sync_copy(dst_hbm.at[pl.ds(base, CHUNK)], dsts[b])

        stage(0, 0)
        g_desc[0] = pltpu.async_copy(hs_hbm.at[src_v0], rows_v0, g0)
        for j in range(NUM_CHUNKS):
            b = j & 1
            nb = 1 - b
            if j + 1 < NUM_CHUNKS:
                if s_desc[nb] is not None:
                    s_desc[nb].wait()
                stage(j + 1, nb)
                g_desc[nb] = pltpu.async_copy(hs_hbm.at[srcs[nb]], rows[nb], gsems[nb])
            g_desc[b].wait()
            s_desc[b] = pltpu.async_copy(
                rows[b], acc_sh.at[dsts[b]], ssems[b], add=True
            )
        for d_ in s_desc:
            if d_ is not None:
                d_.wait()
        plsc.subcore_barrier()
        pltpu.sync_copy(acc_sh.at[pl.ds(row0, CHUNK)], rows_v0)
        pltpu.sync_copy(rows_v0, out_hbm.at[cid, pl.ds(row0, CHUNK)])
        pltpu.sync_copy(
            acc_sh.at[pl.ds(row0 + CHUNK, ROWS_TAIL)], rows_v1.at[pl.ds(0, ROWS_TAIL)]
        )
        pltpu.sync_copy(
            rows_v1.at[pl.ds(0, ROWS_TAIL)], out_hbm.at[cid, pl.ds(row0 + CHUNK, ROWS_TAIL)]
        )

    return scatter_kernel


_BN = 5000
_GRID = N // _BN


def _mm_body(x_ref, w_ref, d0_ref, d1_ref, h_ref, hs_ref):
    h = jnp.dot(x_ref[...], w_ref[...], preferred_element_type=jnp.float32)
    deg = d0_ref[...] + d1_ref[...] + 1.0
    h_ref[...] = h
    hs_ref[...] = h * lax.rsqrt(deg)


def _tc_matmul(x, w1, d0, d1):
    bn = 2000
    return pl.pallas_call(
        _mm_body,
        grid=(N // bn,),
        in_specs=[
            pl.BlockSpec((bn, D_IN), lambda i: (i, 0)),
            pl.BlockSpec((D_IN, D_HID), lambda i: (0, 0)),
            pl.BlockSpec((bn, 1), lambda i: (i, 0)),
            pl.BlockSpec((bn, 1), lambda i: (i, 0)),
        ],
        out_specs=[
            pl.BlockSpec((bn, D_HID), lambda i: (i, 0)),
            pl.BlockSpec((bn, D_HID), lambda i: (i, 0)),
        ],
        out_shape=[
            jax.ShapeDtypeStruct((N, D_HID), jnp.float32),
            jax.ShapeDtypeStruct((N, D_HID), jnp.float32),
        ],
    )(x, w1, d0, d1)


def _mid_body(h1_ref, p0_ref, p1_ref, d0_ref, d1_ref, b1_ref, w2_ref, h2_ref, hs2_ref):
    deg = d0_ref[...] + d1_ref[...] + 1.0
    invd = 1.0 / deg
    dis = lax.rsqrt(deg)
    pre = (p0_ref[...] + p1_ref[...]) * dis + h1_ref[...] * invd + b1_ref[...]
    a = jnp.maximum(pre, 0.0)
    h2 = jnp.dot(a, w2_ref[...], preferred_element_type=jnp.float32)
    h2_ref[...] = h2
    hs2_ref[...] = h2 * dis


def _tc_mid(h1, p0, p1, d0, d1, b1_2d, w2p):
    return pl.pallas_call(
        _mid_body,
        grid=(_GRID,),
        in_specs=[
            pl.BlockSpec((_BN, D_HID), lambda i: (i, 0)),
            pl.BlockSpec((_BN, D_HID), lambda i: (i, 0)),
            pl.BlockSpec((_BN, D_HID), lambda i: (i, 0)),
            pl.BlockSpec((_BN, 1), lambda i: (i, 0)),
            pl.BlockSpec((_BN, 1), lambda i: (i, 0)),
            pl.BlockSpec((1, D_HID), lambda i: (0, 0)),
            pl.BlockSpec((D_HID, 8), lambda i: (0, 0)),
        ],
        out_specs=[
            pl.BlockSpec((_BN, 8), lambda i: (i, 0)),
            pl.BlockSpec((_BN, 8), lambda i: (i, 0)),
        ],
        out_shape=[
            jax.ShapeDtypeStruct((N, 8), jnp.float32),
            jax.ShapeDtypeStruct((N, 8), jnp.float32),
        ],
    )(h1, p0, p1, d0, d1, b1_2d, w2p)


def _final_body(h2_ref, q0_ref, q1_ref, d0_ref, d1_ref, b2_ref, o_ref):
    deg = d0_ref[...] + d1_ref[...] + 1.0
    invd = 1.0 / deg
    dis = lax.rsqrt(deg)
    l = (q0_ref[...] + q1_ref[...]) * dis + h2_ref[...] * invd + b2_ref[...]
    col = lax.broadcasted_iota(jnp.int32, l.shape, 1)
    valid = col < 7
    lm = jnp.where(valid, l, -1e30)
    m = jnp.max(lm, axis=1, keepdims=True)
    e = jnp.where(valid, jnp.exp(l - m), 0.0)
    s = jnp.sum(e, axis=1, keepdims=True)
    out = l - m - jnp.log(s)
    o_ref[...] = out[:, :7]


def _tc_final(h2, q0, q1, d0, d1, b2_2d):
    return pl.pallas_call(
        _final_body,
        grid=(_GRID,),
        in_specs=[
            pl.BlockSpec((_BN, 8), lambda i: (i, 0)),
            pl.BlockSpec((_BN, 8), lambda i: (i, 0)),
            pl.BlockSpec((_BN, 8), lambda i: (i, 0)),
            pl.BlockSpec((_BN, 1), lambda i: (i, 0)),
            pl.BlockSpec((_BN, 1), lambda i: (i, 0)),
            pl.BlockSpec((1, 8), lambda i: (0, 0)),
        ],
        out_specs=pl.BlockSpec((_BN, 7), lambda i: (i, 0)),
        out_shape=jax.ShapeDtypeStruct((N, 7), jnp.float32),
    )(h2, q0, q1, d0, d1, b2_2d)


def kernel(x, edge_index, W1, b1, W2, b2):
    edge_index = edge_index.astype(jnp.int32)
    src = edge_index[0]
    dst = edge_index[1]

    zeros1 = jnp.zeros((NP,), jnp.float32)
    zeros16 = jnp.zeros((CHUNK, D_HID), jnp.float32)
    zeros8 = jnp.zeros((CHUNK, 8), jnp.float32)
    ones_c = jnp.ones((CHUNK,), jnp.float32)

    degs = _make_deg_kernel()(dst, zeros1, ones_c).reshape(NUM_CORES, NP)
    d0 = degs[0, :N][:, None]
    d1 = degs[1, :N][:, None]

    h1, hs1 = _tc_matmul(x, W1, d0, d1)

    p = _make_scatter_kernel(D_HID)(hs1, src, dst, zeros16)
    p0 = p[0, :N]
    p1 = p[1, :N]

    w2p = jnp.concatenate([W2, jnp.zeros((D_HID, 1), jnp.float32)], axis=1)
    b1_2d = b1[None, :]
    h2, hs2 = _tc_mid(h1, p0, p1, d0, d1, b1_2d, w2p)

    q = _make_scatter_kernel(8)(hs2, src, dst, zeros8)
    q0 = q[0, :N]
    q1 = q[1, :N]

    b2_2d = jnp.concatenate([b2, jnp.zeros((1,), jnp.float32)])[None, :]
    return _tc_final(h2, q0, q1, d0, d1, b2_2d)
